# Initial kernel scaffold; baseline (speedup 1.0000x reference)
import functools

import jax
import jax.numpy as jnp
from jax import lax
from jax.experimental import pallas as pl
from jax.experimental.pallas import tpu as pltpu
from jax.experimental.pallas import tpu_sc as plsc

NC, NS, L = 2, 16, 16
NW = NC * NS
D = 128
CH = 128

N_WORDS = 10000
N_WORDS_PAD = 10240
WORD_ROWS_PER_TILE = N_WORDS_PAD // NS

N_DOCS = 16384
DOC_HALF = N_DOCS // NC
DOC_ACC = 8320
DOC_ROWS_PER_TILE = DOC_ACC // NS

ZROWS = 640


def _mesh():
    return plsc.VectorSubcoreMesh(core_axis_name="c", subcore_axis_name="s")


def _sc_spmm_words(row, col, val, h, zeros):
    epad = row.shape[0]
    e_per_tile = epad // NW
    n_chunks = e_per_tile // CH

    @functools.partial(
        pl.kernel,
        out_type=jax.ShapeDtypeStruct((NC, N_WORDS_PAD, D), jnp.float32),
        mesh=_mesh(),
        scratch_types=[
            pltpu.VMEM((CH,), jnp.int32),
            pltpu.VMEM((CH,), jnp.int32),
            pltpu.SMEM((CH,), jnp.float32),
            pltpu.VMEM((CH, D), jnp.float32),
            pltpu.VMEM_SHARED((N_WORDS_PAD, D), jnp.float32),
            pltpu.SemaphoreType.DMA,
        ],
    )
    def k(row_hbm, col_hbm, val_hbm, h_hbm, z_hbm, out_hbm,
          rowv, colv, vals, rows, acc, sem):
        cid = lax.axis_index("c")
        sid = lax.axis_index("s")
        wid = cid * NS + sid
        pltpu.sync_copy(z_hbm, acc.at[pl.ds(sid * WORD_ROWS_PER_TILE,
                                            WORD_ROWS_PER_TILE)])
        plsc.subcore_barrier()
        tile_base = wid * e_per_tile

        def chunk(c, carry):
            base = tile_base + c * CH
            pltpu.sync_copy(col_hbm.at[pl.ds(base, CH)], colv)
            pltpu.sync_copy(row_hbm.at[pl.ds(base, CH)], rowv)
            pltpu.sync_copy(val_hbm.at[pl.ds(base, CH)], vals)
            pltpu.async_copy(h_hbm.at[colv], rows, sem).wait()

            def scale(j, c2):
                v = vals[j]
                for dd in range(D // L):
                    sl = pl.ds(dd * L, L)
                    rows[j, sl] = rows[j, sl] * v
                return c2
            lax.fori_loop(0, CH, scale, 0, unroll=2)
            pltpu.sync_copy(rows, acc.at[rowv], add=True)
            return carry
        lax.fori_loop(0, n_chunks, chunk, 0)
        plsc.subcore_barrier()
        for b in range(WORD_ROWS_PER_TILE // CH):
            r0 = sid * WORD_ROWS_PER_TILE + b * CH
            pltpu.sync_copy(acc.at[pl.ds(r0, CH)], out_hbm.at[cid, pl.ds(r0, CH)])

    return k(row, col, val, h, zeros)


def _sc_spmm_docs(doc, word, val, h, zeros):
    ex = doc.shape[0]
    e_per_tile = ex // NS
    n_chunks = e_per_tile // CH

    @functools.partial(
        pl.kernel,
        out_type=jax.ShapeDtypeStruct((N_DOCS, D), jnp.float32),
        mesh=_mesh(),
        scratch_types=[
            pltpu.VMEM((CH,), jnp.int32),
            pltpu.VMEM((CH,), jnp.int32),
            pltpu.SMEM((CH,), jnp.float32),
            pltpu.VMEM((CH, D), jnp.float32),
            pltpu.VMEM_SHARED((DOC_ACC, D), jnp.float32),
            pltpu.SemaphoreType.DMA,
        ],
    )
    def k(doc_hbm, word_hbm, val_hbm, h_hbm, z_hbm, out_hbm,
          docv, wordv, vals, rows, acc, sem):
        cid = lax.axis_index("c")
        sid = lax.axis_index("s")
        pltpu.sync_copy(z_hbm.at[pl.ds(0, DOC_ROWS_PER_TILE)],
                        acc.at[pl.ds(sid * DOC_ROWS_PER_TILE,
                                     DOC_ROWS_PER_TILE)])
        plsc.subcore_barrier()
        tile_base = sid * e_per_tile
        doc_base = cid * DOC_HALF

        def chunk(c, carry):
            base = tile_base + c * CH
            pltpu.sync_copy(word_hbm.at[pl.ds(base, CH)], wordv)
            pltpu.sync_copy(doc_hbm.at[pl.ds(base, CH)], docv)
            pltpu.sync_copy(val_hbm.at[pl.ds(base, CH)], vals)
            pltpu.async_copy(h_hbm.at[wordv], rows, sem).wait()

            def remap(g, c2):
                sl = pl.ds(g * L, L)
                dv = docv[sl] - doc_base
                ok = (dv >= 0) & (dv < DOC_HALF)
                docv[sl] = jnp.where(ok, dv, DOC_HALF)
                return c2
            lax.fori_loop(0, CH // L, remap, 0, unroll=True)

            def scale(j, c2):
                v = vals[j]
                for dd in range(D // L):
                    sl = pl.ds(dd * L, L)
                    rows[j, sl] = rows[j, sl] * v
                return c2
            lax.fori_loop(0, CH, scale, 0, unroll=2)
            pltpu.sync_copy(rows, acc.at[docv], add=True)
            return carry
        lax.fori_loop(0, n_chunks, chunk, 0)
        plsc.subcore_barrier()
        rows_out_per_tile = DOC_HALF // NS
        for b in range(rows_out_per_tile // CH):
            r0 = sid * rows_out_per_tile + b * CH
            pltpu.sync_copy(acc.at[pl.ds(r0, CH)],
                            out_hbm.at[pl.ds(cid * DOC_HALF + r0, CH)])

    return k(doc, word, val, h, zeros)


def _tc_dense1(p0, p1, w1t):
    m = p0.shape[0]
    bm_ = 256

    def body(a_ref, b_ref, w_ref, o_ref):
        x = a_ref[...] + b_ref[...]
        o_ref[...] = jnp.maximum(
            jnp.dot(x, w_ref[...], preferred_element_type=jnp.float32), 0.0)

    return pl.pallas_call(
        body,
        grid=(m // bm_,),
        in_specs=[pl.BlockSpec((bm_, D), lambda i: (i, 0)),
                  pl.BlockSpec((bm_, D), lambda i: (i, 0)),
                  pl.BlockSpec((D, D), lambda i: (0, 0))],
        out_specs=pl.BlockSpec((bm_, D), lambda i: (i, 0)),
        out_shape=jax.ShapeDtypeStruct((m, D), jnp.float32),
    )(p0, p1, w1t)


def _tc_dense2(p0, p1, emb, w2t, g, b):
    m = p0.shape[0]
    bm_ = 256

    def body(a_ref, b_ref, e_ref, w_ref, g_ref, bt_ref, o_ref):
        x = a_ref[...] + b_ref[...]
        t = jnp.maximum(
            jnp.dot(x, w_ref[...], preferred_element_type=jnp.float32), 0.0)
        h0 = e_ref[...]
        h = (1.0 - 0.7) * h0 + 0.7 * t
        mu = jnp.mean(h, axis=-1, keepdims=True)
        var = jnp.mean((h - mu) ** 2, axis=-1, keepdims=True)
        wh = (h - mu) / jnp.sqrt(var + 1e-5) * g_ref[...] + bt_ref[...]
        o_ref[...] = wh + h0

    return pl.pallas_call(
        body,
        grid=(m // bm_,),
        in_specs=[pl.BlockSpec((bm_, D), lambda i: (i, 0)),
                  pl.BlockSpec((bm_, D), lambda i: (i, 0)),
                  pl.BlockSpec((bm_, D), lambda i: (i, 0)),
                  pl.BlockSpec((D, D), lambda i: (0, 0)),
                  pl.BlockSpec((1, D), lambda i: (0, 0)),
                  pl.BlockSpec((1, D), lambda i: (0, 0))],
        out_specs=pl.BlockSpec((bm_, D), lambda i: (i, 0)),
        out_shape=jax.ShapeDtypeStruct((m, D), jnp.float32),
    )(p0, p1, emb, w2t, g, b)


def _tc_dense3(dh, wmt, bm_row, wct, bc_row):
    m = dh.shape[0]
    bm_ = 512

    def body(d_ref, wm_ref, b1_ref, wc_ref, b2_ref, o_ref):
        t = jnp.maximum(
            jnp.dot(d_ref[...], wm_ref[...], preferred_element_type=jnp.float32)
            + b1_ref[...], 0.0)
        o_ref[...] = (jnp.dot(t, wc_ref[...], preferred_element_type=jnp.float32)
                      + b2_ref[...])

    return pl.pallas_call(
        body,
        grid=(m // bm_,),
        in_specs=[pl.BlockSpec((bm_, D), lambda i: (i, 0)),
                  pl.BlockSpec((D, D), lambda i: (0, 0)),
                  pl.BlockSpec((1, D), lambda i: (0, 0)),
                  pl.BlockSpec((D, D), lambda i: (0, 0)),
                  pl.BlockSpec((1, D), lambda i: (0, 0))],
        out_specs=pl.BlockSpec((bm_, D), lambda i: (i, 0)),
        out_shape=jax.ShapeDtypeStruct((m, D), jnp.float32),
    )(dh, wmt, bm_row, wct, bc_row)


def kernel(a_indices, a_values, x_doc_idx, x_word_idx, x_values,
           emb, W1, W2, ln_gamma, ln_beta, Wm, bm, Wc, bc):
    row, col = a_indices[0], a_indices[1]
    ea = row.shape[0]
    eap = ((ea + CH * NW - 1) // (CH * NW)) * (CH * NW)
    pad = eap - ea
    rowp = jnp.concatenate([row, jnp.zeros((pad,), jnp.int32)])
    colp = jnp.concatenate([col, jnp.zeros((pad,), jnp.int32)])
    valp = jnp.concatenate([a_values, jnp.zeros((pad,), jnp.float32)])
    embp = jnp.concatenate(
        [emb, jnp.zeros((N_WORDS_PAD - emb.shape[0], D), jnp.float32)], axis=0)
    zeros = jnp.zeros((ZROWS, D), jnp.float32)

    p = _sc_spmm_words(rowp, colp, valp, embp, zeros)
    h1 = _tc_dense1(p[0], p[1], W1.T)
    p2 = _sc_spmm_words(rowp, colp, valp, h1, zeros)
    z = _tc_dense2(p2[0], p2[1], embp, W2.T,
                   ln_gamma.reshape(1, D), ln_beta.reshape(1, D))
    dh = _sc_spmm_docs(x_doc_idx, x_word_idx, x_values, z, zeros)

    wct = jnp.zeros((D, D), jnp.float32).at[:, :2].set(Wc.T)
    bcp = jnp.zeros((1, D), jnp.float32).at[0, :2].set(bc)
    out = _tc_dense3(dh, Wm.T, bm.reshape(1, D), wct, bcp)
    return out[:, :2]


# speedup vs baseline: 5.1994x; 5.1994x over previous
import functools

import jax
import jax.numpy as jnp
from jax import lax
from jax.experimental import pallas as pl
from jax.experimental.pallas import tpu as pltpu
from jax.experimental.pallas import tpu_sc as plsc

NC, NS, L = 2, 16, 16
NW = NC * NS
D = 128
CH = 128

N_WORDS = 10000
N_WORDS_PAD = 10240
WORD_ROWS_PER_TILE = N_WORDS_PAD // NS

N_DOCS = 16384
DOC_HALF = N_DOCS // NC
DOC_ACC = 8320
DOC_ROWS_PER_TILE = DOC_ACC // NS

ZROWS = 640


def _mesh():
    return plsc.VectorSubcoreMesh(core_axis_name="c", subcore_axis_name="s")


def _sc_spmm_words(row, col, val, h, zeros):
    epad = row.shape[0]
    e_per_tile = epad // NW
    n_chunks = e_per_tile // CH

    @functools.partial(
        pl.kernel,
        out_type=jax.ShapeDtypeStruct((NC, N_WORDS_PAD, D), jnp.float32),
        mesh=_mesh(),
        scratch_types=[
            pltpu.VMEM((CH,), jnp.int32),
            pltpu.VMEM((CH,), jnp.int32),
            pltpu.VMEM((CH,), jnp.float32),
            pltpu.VMEM((CH, D), jnp.float32),
            pltpu.VMEM_SHARED((N_WORDS_PAD, D), jnp.float32),
            pltpu.SemaphoreType.DMA,
        ],
    )
    def k(row_hbm, col_hbm, val_hbm, h_hbm, z_hbm, out_hbm,
          rowv, colv, vals, rows, acc, sem):
        cid = lax.axis_index("c")
        sid = lax.axis_index("s")
        wid = cid * NS + sid
        pltpu.sync_copy(z_hbm, acc.at[pl.ds(sid * WORD_ROWS_PER_TILE,
                                            WORD_ROWS_PER_TILE)])
        plsc.subcore_barrier()
        tile_base = wid * e_per_tile

        def chunk(c, carry):
            base = tile_base + c * CH
            pltpu.sync_copy(col_hbm.at[pl.ds(base, CH)], colv)
            pltpu.sync_copy(row_hbm.at[pl.ds(base, CH)], rowv)
            pltpu.sync_copy(val_hbm.at[pl.ds(base, CH)], vals)
            pltpu.async_copy(h_hbm.at[colv], rows, sem).wait()

            def scale(g, c2):
                v16 = vals[pl.ds(g * L, L)]
                for j in range(L):
                    r = g * L + j
                    vj = v16[j]
                    for dd in range(D // L):
                        sl = pl.ds(dd * L, L)
                        rows[r, sl] = rows[r, sl] * vj
                return c2
            lax.fori_loop(0, CH // L, scale, 0)
            pltpu.sync_copy(rows, acc.at[rowv], add=True)
            return carry
        lax.fori_loop(0, n_chunks, chunk, 0)
        plsc.subcore_barrier()
        for b in range(WORD_ROWS_PER_TILE // CH):
            r0 = sid * WORD_ROWS_PER_TILE + b * CH
            pltpu.sync_copy(acc.at[pl.ds(r0, CH)], out_hbm.at[cid, pl.ds(r0, CH)])

    return k(row, col, val, h, zeros)


def _sc_spmm_docs(doc, word, val, h, zeros):
    ex = doc.shape[0]
    e_per_tile = ex // NS
    n_chunks = e_per_tile // CH

    @functools.partial(
        pl.kernel,
        out_type=jax.ShapeDtypeStruct((N_DOCS, D), jnp.float32),
        mesh=_mesh(),
        scratch_types=[
            pltpu.VMEM((CH,), jnp.int32),
            pltpu.VMEM((CH,), jnp.int32),
            pltpu.VMEM((CH,), jnp.float32),
            pltpu.VMEM((CH, D), jnp.float32),
            pltpu.VMEM_SHARED((DOC_ACC, D), jnp.float32),
            pltpu.SemaphoreType.DMA,
        ],
    )
    def k(doc_hbm, word_hbm, val_hbm, h_hbm, z_hbm, out_hbm,
          docv, wordv, vals, rows, acc, sem):
        cid = lax.axis_index("c")
        sid = lax.axis_index("s")
        pltpu.sync_copy(z_hbm.at[pl.ds(0, DOC_ROWS_PER_TILE)],
                        acc.at[pl.ds(sid * DOC_ROWS_PER_TILE,
                                     DOC_ROWS_PER_TILE)])
        plsc.subcore_barrier()
        tile_base = sid * e_per_tile
        doc_base = cid * DOC_HALF

        def chunk(c, carry):
            base = tile_base + c * CH
            pltpu.sync_copy(word_hbm.at[pl.ds(base, CH)], wordv)
            pltpu.sync_copy(doc_hbm.at[pl.ds(base, CH)], docv)
            pltpu.sync_copy(val_hbm.at[pl.ds(base, CH)], vals)
            pltpu.async_copy(h_hbm.at[wordv], rows, sem).wait()

            def remap(g, c2):
                sl = pl.ds(g * L, L)
                dv = docv[sl] - doc_base
                ok = (dv >= 0) & (dv < DOC_HALF)
                docv[sl] = jnp.where(ok, dv, DOC_HALF)
                return c2
            lax.fori_loop(0, CH // L, remap, 0, unroll=True)

            def scale(g, c2):
                v16 = vals[pl.ds(g * L, L)]
                for j in range(L):
                    r = g * L + j
                    vj = v16[j]
                    for dd in range(D // L):
                        sl = pl.ds(dd * L, L)
                        rows[r, sl] = rows[r, sl] * vj
                return c2
            lax.fori_loop(0, CH // L, scale, 0)
            pltpu.sync_copy(rows, acc.at[docv], add=True)
            return carry
        lax.fori_loop(0, n_chunks, chunk, 0)
        plsc.subcore_barrier()
        rows_out_per_tile = DOC_HALF // NS
        for b in range(rows_out_per_tile // CH):
            r0 = sid * rows_out_per_tile + b * CH
            pltpu.sync_copy(acc.at[pl.ds(r0, CH)],
                            out_hbm.at[pl.ds(cid * DOC_HALF + r0, CH)])

    return k(doc, word, val, h, zeros)


def _tc_dense1(p0, p1, w1t):
    m = p0.shape[0]
    bm_ = 256

    def body(a_ref, b_ref, w_ref, o_ref):
        x = a_ref[...] + b_ref[...]
        o_ref[...] = jnp.maximum(
            jnp.dot(x, w_ref[...], preferred_element_type=jnp.float32), 0.0)

    return pl.pallas_call(
        body,
        grid=(m // bm_,),
        in_specs=[pl.BlockSpec((bm_, D), lambda i: (i, 0)),
                  pl.BlockSpec((bm_, D), lambda i: (i, 0)),
                  pl.BlockSpec((D, D), lambda i: (0, 0))],
        out_specs=pl.BlockSpec((bm_, D), lambda i: (i, 0)),
        out_shape=jax.ShapeDtypeStruct((m, D), jnp.float32),
    )(p0, p1, w1t)


def _tc_dense2(p0, p1, emb, w2t, g, b):
    m = p0.shape[0]
    bm_ = 256

    def body(a_ref, b_ref, e_ref, w_ref, g_ref, bt_ref, o_ref):
        x = a_ref[...] + b_ref[...]
        t = jnp.maximum(
            jnp.dot(x, w_ref[...], preferred_element_type=jnp.float32), 0.0)
        h0 = e_ref[...]
        h = (1.0 - 0.7) * h0 + 0.7 * t
        mu = jnp.mean(h, axis=-1, keepdims=True)
        var = jnp.mean((h - mu) ** 2, axis=-1, keepdims=True)
        wh = (h - mu) / jnp.sqrt(var + 1e-5) * g_ref[...] + bt_ref[...]
        o_ref[...] = wh + h0

    return pl.pallas_call(
        body,
        grid=(m // bm_,),
        in_specs=[pl.BlockSpec((bm_, D), lambda i: (i, 0)),
                  pl.BlockSpec((bm_, D), lambda i: (i, 0)),
                  pl.BlockSpec((bm_, D), lambda i: (i, 0)),
                  pl.BlockSpec((D, D), lambda i: (0, 0)),
                  pl.BlockSpec((1, D), lambda i: (0, 0)),
                  pl.BlockSpec((1, D), lambda i: (0, 0))],
        out_specs=pl.BlockSpec((bm_, D), lambda i: (i, 0)),
        out_shape=jax.ShapeDtypeStruct((m, D), jnp.float32),
    )(p0, p1, emb, w2t, g, b)


def _tc_dense3(dh, wmt, bm_row, wct, bc_row):
    m = dh.shape[0]
    bm_ = 512

    def body(d_ref, wm_ref, b1_ref, wc_ref, b2_ref, o_ref):
        t = jnp.maximum(
            jnp.dot(d_ref[...], wm_ref[...], preferred_element_type=jnp.float32)
            + b1_ref[...], 0.0)
        o_ref[...] = (jnp.dot(t, wc_ref[...], preferred_element_type=jnp.float32)
                      + b2_ref[...])

    return pl.pallas_call(
        body,
        grid=(m // bm_,),
        in_specs=[pl.BlockSpec((bm_, D), lambda i: (i, 0)),
                  pl.BlockSpec((D, D), lambda i: (0, 0)),
                  pl.BlockSpec((1, D), lambda i: (0, 0)),
                  pl.BlockSpec((D, D), lambda i: (0, 0)),
                  pl.BlockSpec((1, D), lambda i: (0, 0))],
        out_specs=pl.BlockSpec((bm_, D), lambda i: (i, 0)),
        out_shape=jax.ShapeDtypeStruct((m, D), jnp.float32),
    )(dh, wmt, bm_row, wct, bc_row)


def kernel(a_indices, a_values, x_doc_idx, x_word_idx, x_values,
           emb, W1, W2, ln_gamma, ln_beta, Wm, bm, Wc, bc):
    row, col = a_indices[0], a_indices[1]
    ea = row.shape[0]
    eap = ((ea + CH * NW - 1) // (CH * NW)) * (CH * NW)
    pad = eap - ea
    rowp = jnp.concatenate([row, jnp.zeros((pad,), jnp.int32)])
    colp = jnp.concatenate([col, jnp.zeros((pad,), jnp.int32)])
    valp = jnp.concatenate([a_values, jnp.zeros((pad,), jnp.float32)])
    embp = jnp.concatenate(
        [emb, jnp.zeros((N_WORDS_PAD - emb.shape[0], D), jnp.float32)], axis=0)
    zeros = jnp.zeros((ZROWS, D), jnp.float32)

    p = _sc_spmm_words(rowp, colp, valp, embp, zeros)
    h1 = _tc_dense1(p[0], p[1], W1.T)
    p2 = _sc_spmm_words(rowp, colp, valp, h1, zeros)
    z = _tc_dense2(p2[0], p2[1], embp, W2.T,
                   ln_gamma.reshape(1, D), ln_beta.reshape(1, D))
    dh = _sc_spmm_docs(x_doc_idx, x_word_idx, x_values, z, zeros)

    wct = jnp.zeros((D, D), jnp.float32).at[:, :2].set(Wc.T)
    bcp = jnp.zeros((1, D), jnp.float32).at[0, :2].set(bc)
    out = _tc_dense3(dh, Wm.T, bm.reshape(1, D), wct, bcp)
    return out[:, :2]


# speedup vs baseline: 5.2141x; 1.0028x over previous
import functools

import jax
import jax.numpy as jnp
from jax import lax
from jax.experimental import pallas as pl
from jax.experimental.pallas import tpu as pltpu
from jax.experimental.pallas import tpu_sc as plsc

NC, NS, L = 2, 16, 16
NW = NC * NS
D = 128
DW = 64
CH = 128

N_WORDS = 10000
N_WORDS_PAD = 10240
WORD_ROWS_PER_TILE = N_WORDS_PAD // NS

N_DOCS = 16384
DOC_ROWS_PER_TILE = N_DOCS // NS

ZROWS = 640


def _mesh():
    return plsc.VectorSubcoreMesh(core_axis_name="c", subcore_axis_name="s")


def _scale_rows(rows, vals, width):
    def scale(g, c2):
        v16 = vals[pl.ds(g * L, L)]
        for j in range(L):
            r = g * L + j
            vj = v16[j]
            for dd in range(width // L):
                sl = pl.ds(dd * L, L)
                rows[r, sl] = rows[r, sl] * vj
        return c2
    lax.fori_loop(0, CH // L, scale, 0)


def _sc_spmm_words(row, col, val, h, zeros):
    epad = row.shape[0]
    e_per_tile = epad // NW
    n_chunks = e_per_tile // CH
    assert n_chunks % 2 == 0

    @functools.partial(
        pl.kernel,
        out_type=jax.ShapeDtypeStruct((NC, N_WORDS_PAD, D), jnp.float32),
        mesh=_mesh(),
        scratch_types=[
            pltpu.VMEM((CH,), jnp.int32), pltpu.VMEM((CH,), jnp.int32),
            pltpu.VMEM((CH,), jnp.int32), pltpu.VMEM((CH,), jnp.int32),
            pltpu.VMEM((CH,), jnp.float32), pltpu.VMEM((CH,), jnp.float32),
            pltpu.VMEM((CH, D), jnp.float32), pltpu.VMEM((CH, D), jnp.float32),
            pltpu.VMEM_SHARED((N_WORDS_PAD, D), jnp.float32),
            pltpu.SemaphoreType.DMA, pltpu.SemaphoreType.DMA,
        ],
    )
    def k(row_hbm, col_hbm, val_hbm, h_hbm, z_hbm, out_hbm,
          rowv0, rowv1, colv0, colv1, vals0, vals1, rows0, rows1,
          acc, sem0, sem1):
        cid = lax.axis_index("c")
        sid = lax.axis_index("s")
        wid = cid * NS + sid
        pltpu.sync_copy(z_hbm, acc.at[pl.ds(sid * WORD_ROWS_PER_TILE,
                                            WORD_ROWS_PER_TILE)])
        plsc.subcore_barrier()
        tile_base = wid * e_per_tile
        bufs = ((rowv0, colv0, vals0, rows0, sem0),
                (rowv1, colv1, vals1, rows1, sem1))

        def load_chunk(c, rowv, colv, vals, rows, sem):
            base = tile_base + c * CH
            pltpu.sync_copy(col_hbm.at[pl.ds(base, CH)], colv)
            pltpu.sync_copy(row_hbm.at[pl.ds(base, CH)], rowv)
            pltpu.sync_copy(val_hbm.at[pl.ds(base, CH)], vals)
            pltpu.async_copy(h_hbm.at[colv], rows, sem)

        for b in range(2):
            load_chunk(b, *bufs[b])

        def step(i, carry):
            for b in range(2):
                c = i * 2 + b
                rowv, colv, vals, rows, sem = bufs[b]
                pltpu.make_async_copy(h_hbm.at[colv], rows, sem).wait()
                _scale_rows(rows, vals, D)
                pltpu.sync_copy(rows, acc.at[rowv], add=True)

                @pl.when(c + 2 < n_chunks)
                def _():
                    load_chunk(c + 2, rowv, colv, vals, rows, sem)
            return carry
        lax.fori_loop(0, n_chunks // 2, step, 0)
        plsc.subcore_barrier()
        for b in range(WORD_ROWS_PER_TILE // CH):
            r0 = sid * WORD_ROWS_PER_TILE + b * CH
            pltpu.sync_copy(acc.at[pl.ds(r0, CH)], out_hbm.at[cid, pl.ds(r0, CH)])

    return k(row, col, val, h, zeros)


def _sc_spmm_docs(doc, word, val, h2, zeros64):
    ex = doc.shape[0]
    e_per_tile = ex // NS
    n_chunks = e_per_tile // CH
    assert n_chunks % 2 == 0
    nh = h2.shape[0] // 2

    @functools.partial(
        pl.kernel,
        out_type=jax.ShapeDtypeStruct((NC, N_DOCS, DW), jnp.float32),
        mesh=_mesh(),
        compiler_params=pltpu.CompilerParams(use_tc_tiling_on_sc=False),
        scratch_types=[
            pltpu.VMEM((CH,), jnp.int32), pltpu.VMEM((CH,), jnp.int32),
            pltpu.VMEM((CH,), jnp.int32), pltpu.VMEM((CH,), jnp.int32),
            pltpu.VMEM((CH,), jnp.float32), pltpu.VMEM((CH,), jnp.float32),
            pltpu.VMEM((CH, DW), jnp.float32), pltpu.VMEM((CH, DW), jnp.float32),
            pltpu.VMEM_SHARED((N_DOCS, DW), jnp.float32),
            pltpu.SemaphoreType.DMA, pltpu.SemaphoreType.DMA,
        ],
    )
    def k(doc_hbm, word_hbm, val_hbm, h_hbm, z_hbm, out_hbm,
          docv0, docv1, wordv0, wordv1, vals0, vals1, rows0, rows1,
          acc, sem0, sem1):
        cid = lax.axis_index("c")
        sid = lax.axis_index("s")
        pltpu.sync_copy(z_hbm, acc.at[pl.ds(sid * DOC_ROWS_PER_TILE,
                                            DOC_ROWS_PER_TILE)])
        plsc.subcore_barrier()
        tile_base = sid * e_per_tile
        row_off = cid * nh
        bufs = ((docv0, wordv0, vals0, rows0, sem0),
                (docv1, wordv1, vals1, rows1, sem1))

        def load_chunk(c, docv, wordv, vals, rows, sem):
            base = tile_base + c * CH
            pltpu.sync_copy(word_hbm.at[pl.ds(base, CH)], wordv)
            pltpu.sync_copy(doc_hbm.at[pl.ds(base, CH)], docv)
            pltpu.sync_copy(val_hbm.at[pl.ds(base, CH)], vals)

            def adj(g, c2):
                sl = pl.ds(g * L, L)
                wordv[sl] = wordv[sl] + row_off
                return c2
            lax.fori_loop(0, CH // L, adj, 0, unroll=True)
            pltpu.async_copy(h_hbm.at[wordv], rows, sem)

        for b in range(2):
            load_chunk(b, *bufs[b])

        def step(i, carry):
            for b in range(2):
                c = i * 2 + b
                docv, wordv, vals, rows, sem = bufs[b]
                pltpu.make_async_copy(h_hbm.at[wordv], rows, sem).wait()
                _scale_rows(rows, vals, DW)
                pltpu.sync_copy(rows, acc.at[docv], add=True)

                @pl.when(c + 2 < n_chunks)
                def _():
                    load_chunk(c + 2, docv, wordv, vals, rows, sem)
            return carry
        lax.fori_loop(0, n_chunks // 2, step, 0)
        plsc.subcore_barrier()
        for b in range(DOC_ROWS_PER_TILE // CH):
            r0 = sid * DOC_ROWS_PER_TILE + b * CH
            pltpu.sync_copy(acc.at[pl.ds(r0, CH)], out_hbm.at[cid, pl.ds(r0, CH)])

    return k(doc, word, val, h2, zeros64)


def _tc_dense1(p0, p1, w1t):
    m = p0.shape[0]
    bm_ = 256

    def body(a_ref, b_ref, w_ref, o_ref):
        x = a_ref[...] + b_ref[...]
        o_ref[...] = jnp.maximum(
            jnp.dot(x, w_ref[...], preferred_element_type=jnp.float32), 0.0)

    return pl.pallas_call(
        body,
        grid=(m // bm_,),
        in_specs=[pl.BlockSpec((bm_, D), lambda i: (i, 0)),
                  pl.BlockSpec((bm_, D), lambda i: (i, 0)),
                  pl.BlockSpec((D, D), lambda i: (0, 0))],
        out_specs=pl.BlockSpec((bm_, D), lambda i: (i, 0)),
        out_shape=jax.ShapeDtypeStruct((m, D), jnp.float32),
    )(p0, p1, w1t)


def _tc_dense2(p0, p1, emb, w2t, g, b):
    m = p0.shape[0]
    bm_ = 256

    def body(a_ref, b_ref, e_ref, w_ref, g_ref, bt_ref, o_ref):
        x = a_ref[...] + b_ref[...]
        t = jnp.maximum(
            jnp.dot(x, w_ref[...], preferred_element_type=jnp.float32), 0.0)
        h0 = e_ref[...]
        h = (1.0 - 0.7) * h0 + 0.7 * t
        mu = jnp.mean(h, axis=-1, keepdims=True)
        var = jnp.mean((h - mu) ** 2, axis=-1, keepdims=True)
        wh = (h - mu) / jnp.sqrt(var + 1e-5) * g_ref[...] + bt_ref[...]
        z = wh + h0
        o_ref[0] = z[:, :DW]
        o_ref[1] = z[:, DW:]

    return pl.pallas_call(
        body,
        grid=(m // bm_,),
        in_specs=[pl.BlockSpec((bm_, D), lambda i: (i, 0)),
                  pl.BlockSpec((bm_, D), lambda i: (i, 0)),
                  pl.BlockSpec((bm_, D), lambda i: (i, 0)),
                  pl.BlockSpec((D, D), lambda i: (0, 0)),
                  pl.BlockSpec((1, D), lambda i: (0, 0)),
                  pl.BlockSpec((1, D), lambda i: (0, 0))],
        out_specs=pl.BlockSpec((2, bm_, DW), lambda i: (0, i, 0)),
        out_shape=jax.ShapeDtypeStruct((2, m, DW), jnp.float32),
    )(p0, p1, emb, w2t, g, b)


def _tc_dense3(d0, d1, wmt, bm_row, wct, bc_row):
    m = d0.shape[0]
    bm_ = 512

    def body(d0_ref, d1_ref, wm_ref, b1_ref, wc_ref, b2_ref, o_ref):
        dh = jnp.concatenate((d0_ref[...], d1_ref[...]), axis=-1)
        t = jnp.maximum(
            jnp.dot(dh, wm_ref[...], preferred_element_type=jnp.float32)
            + b1_ref[...], 0.0)
        o_ref[...] = (jnp.dot(t, wc_ref[...], preferred_element_type=jnp.float32)
                      + b2_ref[...])

    return pl.pallas_call(
        body,
        grid=(m // bm_,),
        in_specs=[pl.BlockSpec((bm_, DW), lambda i: (i, 0)),
                  pl.BlockSpec((bm_, DW), lambda i: (i, 0)),
                  pl.BlockSpec((D, D), lambda i: (0, 0)),
                  pl.BlockSpec((1, D), lambda i: (0, 0)),
                  pl.BlockSpec((D, D), lambda i: (0, 0)),
                  pl.BlockSpec((1, D), lambda i: (0, 0))],
        out_specs=pl.BlockSpec((bm_, D), lambda i: (i, 0)),
        out_shape=jax.ShapeDtypeStruct((m, D), jnp.float32),
    )(d0, d1, wmt, bm_row, wct, bc_row)


def kernel(a_indices, a_values, x_doc_idx, x_word_idx, x_values,
           emb, W1, W2, ln_gamma, ln_beta, Wm, bm, Wc, bc):
    row, col = a_indices[0], a_indices[1]
    ea = row.shape[0]
    eap = ((ea + 2 * CH * NW - 1) // (2 * CH * NW)) * (2 * CH * NW)
    pad = eap - ea
    rowp = jnp.concatenate([row, jnp.zeros((pad,), jnp.int32)])
    colp = jnp.concatenate([col, jnp.zeros((pad,), jnp.int32)])
    valp = jnp.concatenate([a_values, jnp.zeros((pad,), jnp.float32)])
    embp = jnp.concatenate(
        [emb, jnp.zeros((N_WORDS_PAD - emb.shape[0], D), jnp.float32)], axis=0)
    zeros = jnp.zeros((ZROWS, D), jnp.float32)
    zeros64 = jnp.zeros((DOC_ROWS_PER_TILE, DW), jnp.float32)

    p = _sc_spmm_words(rowp, colp, valp, embp, zeros)
    h1 = _tc_dense1(p[0], p[1], W1.T)
    p2 = _sc_spmm_words(rowp, colp, valp, h1, zeros)
    z2 = _tc_dense2(p2[0], p2[1], embp, W2.T,
                    ln_gamma.reshape(1, D), ln_beta.reshape(1, D))
    zs = z2.reshape(2 * N_WORDS_PAD, DW)
    dh = _sc_spmm_docs(x_doc_idx, x_word_idx, x_values, zs, zeros64)

    wct = jnp.zeros((D, D), jnp.float32).at[:, :2].set(Wc.T)
    bcp = jnp.zeros((1, D), jnp.float32).at[0, :2].set(bc)
    out = _tc_dense3(dh[0], dh[1], Wm.T, bm.reshape(1, D), wct, bcp)
    return out[:, :2]
